# Initial kernel scaffold; baseline (speedup 1.0000x reference)
import jax
import jax.numpy as jnp
from jax.experimental import pallas as pl


def kernel(ids, cos_0, cos_1, cos_2, sin_0, sin_1, sin_2):
    raise NotImplementedError("write your pallas kernel here")


# speedup vs baseline: 4.3377x; 4.3377x over previous
import functools

import jax
import jax.numpy as jnp
from jax import lax
from jax.experimental import pallas as pl
from jax.experimental.pallas import tpu as pltpu
from jax.experimental.pallas import tpu_sc as plsc

N_IDS = 16384
NUM_CORES = 2
NUM_SUBCORES = 16
NUM_WORKERS = NUM_CORES * NUM_SUBCORES
ROWS_PER_WORKER = N_IDS // NUM_WORKERS

GATHER_WIDTHS = (16, 64, 64)
OUT_D = 128

CONCAT_ROWS = 2048


def _sc_gather(ids_by_axis, tables):
    b = ROWS_PER_WORKER
    mesh = plsc.VectorSubcoreMesh(core_axis_name="c", subcore_axis_name="s")

    scratch = [pltpu.VMEM((b,), jnp.int32) for _ in range(3)]
    scratch += [pltpu.VMEM((b, w), jnp.float32) for w in GATHER_WIDTHS]
    scratch += [pltpu.SemaphoreType.DMA]

    @functools.partial(
        pl.kernel,
        out_type=tuple(
            jax.ShapeDtypeStruct((N_IDS, w), jnp.float32)
            for w in GATHER_WIDTHS
        ),
        mesh=mesh,
        scratch_types=scratch,
        compiler_params=pltpu.CompilerParams(use_tc_tiling_on_sc=False),
    )
    def body(ids0_hbm, ids1_hbm, ids2_hbm, w0, w1, w2,
             o0, o1, o2, i0, i1, i2, b0, b1, b2, sem):
        wid = lax.axis_index("s") * NUM_CORES + lax.axis_index("c")
        base = wid * b
        idxs = (i0, i1, i2)
        for ax, ids_hbm in enumerate((ids0_hbm, ids1_hbm, ids2_hbm)):
            pltpu.sync_copy(ids_hbm.at[pl.ds(base, b)], idxs[ax])
        copies = []
        for t, buf, idx in zip((w0, w1, w2), (b0, b1, b2), idxs):
            copies.append(pltpu.async_copy(t.at[idx], buf, sem))
        for cp in copies:
            cp.wait()
        for buf, out in zip((b0, b1, b2), (o0, o1, o2)):
            pltpu.sync_copy(buf, out.at[pl.ds(base, b), :])

    return body(*ids_by_axis, *tables)


def _tc_shuffle(parts):
    def body(g0, g1, g2, out_ref):
        out_ref[...] = jnp.concatenate(
            [
                g0[:, 0:8],
                g1[:, 0:28],
                g2[:, 0:28],
                g0[:, 8:16],
                g1[:, 28:56],
                g2[:, 28:56],
            ],
            axis=-1,
        )

    grid = (N_IDS // CONCAT_ROWS,)
    in_specs = [
        pl.BlockSpec((CONCAT_ROWS, w), lambda i: (i, 0))
        for w in GATHER_WIDTHS
    ]
    return pl.pallas_call(
        body,
        out_shape=jax.ShapeDtypeStruct((N_IDS, OUT_D), jnp.float32),
        grid=grid,
        in_specs=in_specs,
        out_specs=pl.BlockSpec((CONCAT_ROWS, OUT_D), lambda i: (i, 0)),
    )(*parts)


def kernel(ids, cos_0, cos_1, cos_2, sin_0, sin_1, sin_2):
    ids_by_axis = (ids[:, 0], ids[:, 1], ids[:, 2])
    pad = jnp.zeros((cos_1.shape[0], 8), jnp.float32)
    tables = (
        jnp.concatenate([cos_0, sin_0], axis=1),
        jnp.concatenate([cos_1, sin_1, pad], axis=1),
        jnp.concatenate([cos_2, sin_2, pad], axis=1),
    )
    parts = _sc_gather(ids_by_axis, tables)
    return _tc_shuffle(parts)


# speedup vs baseline: 4.3395x; 1.0004x over previous
import functools

import jax
import jax.numpy as jnp
from jax import lax
from jax.experimental import pallas as pl
from jax.experimental.pallas import tpu as pltpu
from jax.experimental.pallas import tpu_sc as plsc

N_IDS = 16384
NUM_CORES = 2
NUM_SUBCORES = 16
NUM_WORKERS = NUM_CORES * NUM_SUBCORES
ROWS_PER_WORKER = N_IDS // NUM_WORKERS

GATHER_WIDTHS = (16, 64, 64)
OUT_D = 128

CONCAT_ROWS = 2048


def _sc_gather(ids_by_axis, tables):
    b = ROWS_PER_WORKER
    mesh = plsc.VectorSubcoreMesh(core_axis_name="c", subcore_axis_name="s")

    scratch = [pltpu.VMEM((b,), jnp.int32) for _ in range(3)]
    scratch += [pltpu.VMEM((b, w), jnp.float32) for w in GATHER_WIDTHS]
    scratch += [pltpu.SemaphoreType.DMA]

    @functools.partial(
        pl.kernel,
        out_type=tuple(
            jax.ShapeDtypeStruct((N_IDS, w), jnp.float32)
            for w in GATHER_WIDTHS
        ),
        mesh=mesh,
        scratch_types=scratch,
        compiler_params=pltpu.CompilerParams(use_tc_tiling_on_sc=False),
    )
    def body(ids0_hbm, ids1_hbm, ids2_hbm, w0, w1, w2,
             o0, o1, o2, i0, i1, i2, b0, b1, b2, sem):
        wid = lax.axis_index("s") * NUM_CORES + lax.axis_index("c")
        base = wid * b
        idxs = (i0, i1, i2)
        for ax, ids_hbm in enumerate((ids0_hbm, ids1_hbm, ids2_hbm)):
            pltpu.sync_copy(ids_hbm.at[pl.ds(base, b)], idxs[ax])
        copies = []
        ns, ch = 4, b // 4
        for t, buf, idx in zip((w0, w1, w2), (b0, b1, b2), idxs):
            for s in range(ns):
                copies.append(pltpu.async_copy(
                    t.at[idx.at[pl.ds(s * ch, ch)]],
                    buf.at[pl.ds(s * ch, ch), :], sem))
        for cp in copies:
            cp.wait()
        for buf, out in zip((b0, b1, b2), (o0, o1, o2)):
            pltpu.sync_copy(buf, out.at[pl.ds(base, b), :])

    return body(*ids_by_axis, *tables)


def _tc_shuffle(parts):
    def body(g0, g1, g2, out_ref):
        out_ref[...] = jnp.concatenate(
            [
                g0[:, 0:8],
                g1[:, 0:28],
                g2[:, 0:28],
                g0[:, 8:16],
                g1[:, 28:56],
                g2[:, 28:56],
            ],
            axis=-1,
        )

    grid = (N_IDS // CONCAT_ROWS,)
    in_specs = [
        pl.BlockSpec((CONCAT_ROWS, w), lambda i: (i, 0))
        for w in GATHER_WIDTHS
    ]
    return pl.pallas_call(
        body,
        out_shape=jax.ShapeDtypeStruct((N_IDS, OUT_D), jnp.float32),
        grid=grid,
        in_specs=in_specs,
        out_specs=pl.BlockSpec((CONCAT_ROWS, OUT_D), lambda i: (i, 0)),
    )(*parts)


def kernel(ids, cos_0, cos_1, cos_2, sin_0, sin_1, sin_2):
    ids_by_axis = (ids[:, 0], ids[:, 1], ids[:, 2])
    pad = jnp.zeros((cos_1.shape[0], 8), jnp.float32)
    tables = (
        jnp.concatenate([cos_0, sin_0], axis=1),
        jnp.concatenate([cos_1, sin_1, pad], axis=1),
        jnp.concatenate([cos_2, sin_2, pad], axis=1),
    )
    parts = _sc_gather(ids_by_axis, tables)
    return _tc_shuffle(parts)
